# Initial kernel scaffold; baseline (speedup 1.0000x reference)
import jax
import jax.numpy as jnp
from jax.experimental import pallas as pl


def kernel(seq, tok_table, pos_table):
    raise NotImplementedError("write your pallas kernel here")


# speedup vs baseline: 1.1308x; 1.1308x over previous
import functools

import jax
import jax.numpy as jnp
from jax import lax
from jax.experimental import pallas as pl
from jax.experimental.pallas import tpu as pltpu
from jax.experimental.pallas import tpu_sc as plsc

NUM_CORES = 2
NUM_SUBCORES = 16
NW = NUM_CORES * NUM_SUBCORES
LANES = 16


@functools.cache
def _make_kernel(B, S, D):
    T = B * S
    s_per_w = S // NW
    mesh = plsc.VectorSubcoreMesh(core_axis_name="c", subcore_axis_name="s")

    @functools.partial(
        pl.kernel,
        out_type=jax.ShapeDtypeStruct((T, D), jnp.float32),
        mesh=mesh,
        scratch_types=[
            pltpu.VMEM((B, s_per_w), jnp.int32),
            pltpu.VMEM((s_per_w, D), jnp.float32),
            pltpu.VMEM((s_per_w, D), jnp.float32),
            pltpu.SemaphoreType.DMA,
        ],
    )
    def k(seq_hbm, tok_hbm, pos_hbm, out_hbm, idx_v, pos_v, buf_v, gsem):
        wid = lax.axis_index("s") * NUM_CORES + lax.axis_index("c")
        s0 = wid * s_per_w
        pltpu.sync_copy(pos_hbm.at[pl.ds(s0, s_per_w)], pos_v)
        for b in range(B):
            pltpu.sync_copy(seq_hbm.at[pl.ds(b * S + s0, s_per_w)], idx_v.at[b])
        for b in range(B):
            pltpu.async_copy(tok_hbm.at[idx_v.at[b]], buf_v, gsem).wait()

            def add_row(r, _):
                for c in range(D // LANES):
                    sl = pl.ds(c * LANES, LANES)
                    plsc.addupdate(buf_v.at[r, sl], pos_v[r, sl])
                return _

            lax.fori_loop(0, s_per_w, add_row, None)
            pltpu.sync_copy(buf_v, out_hbm.at[pl.ds(b * S + s0, s_per_w)])

    return k


def kernel(seq, tok_table, pos_table):
    B, S = seq.shape
    V, D = tok_table.shape
    k = _make_kernel(B, S, D)
    out = k(seq.reshape(B * S).astype(jnp.int32), tok_table, pos_table)
    return out.reshape(B, S, D)
